# Initial kernel scaffold; baseline (speedup 1.0000x reference)
import jax
import jax.numpy as jnp
from jax.experimental import pallas as pl


def kernel(queries, keys, values, top_num):
    raise NotImplementedError("write your pallas kernel here")


# speedup vs baseline: 6.9106x; 6.9106x over previous
import functools

import jax
import jax.numpy as jnp
from jax import lax
from jax.experimental import pallas as pl
from jax.experimental.pallas import tpu as pltpu
from jax.experimental.pallas import tpu_sc as plsc

MEM = 1_000_000
H = 64
B = 1024
K = 8
LBLK = 1024
NB = 977
VALID_LAST = MEM - (NB - 1) * LBLK
CHUNK = 128
CPB = LBLK // CHUNK
NCHUNK = NB * CPB
QG = 8
NGC = B // QG
EPS = 1e-6
IBIG = 2**31 - 1

NEG = float("-inf")


def _cmax_body(qn_ref, keys_ref, nmax_ref, out_ref, kn_ref):
    i = pl.program_id(0)
    kn = keys_ref[...] / nmax_ref[...]
    kn_ref[...] = kn
    simsT = lax.dot_general(kn, qn_ref[...], (((1,), (1,)), ((), ())),
                            preferred_element_type=jnp.float32)

    def cmax(s):
        parts = [jnp.max(s[j * CHUNK:(j + 1) * CHUNK, :], axis=0, keepdims=True)
                 for j in range(CPB)]
        return jnp.concatenate(parts, axis=0)

    out_ref[...] = cmax(simsT)

    @pl.when(i == NB - 1)
    def _():
        row = lax.broadcasted_iota(jnp.int32, (LBLK, B), 0)
        out_ref[...] = cmax(jnp.where(row < VALID_LAST, simsT, NEG))


def _topchunk_body(cmax_ref, out_ref):
    v = cmax_ref[...]
    cidx = lax.broadcasted_iota(jnp.int32, (NCHUNK, 128), 0)
    for j in range(K):
        m = jnp.max(v, axis=0, keepdims=True)
        sel = jnp.min(jnp.where(v == m, cidx, IBIG), axis=0, keepdims=True)
        out_ref[j:j + 1, :] = sel
        v = jnp.where(cidx == sel, NEG, v)


def _cand_body(cid_ref, qn_ref, *refs):
    kn_refs = refs[:QG * K]
    out_ref = refs[QG * K]
    g = pl.program_id(0)

    kcat = jnp.concatenate([r[...] for r in kn_refs], axis=0)
    full = lax.dot_general(qn_ref[...], kcat, (((1,), (1,)), ((), ())),
                           preferred_element_type=jnp.float32)
    span = K * CHUNK
    rows = [full[qi:qi + 1, qi * span:(qi + 1) * span] for qi in range(QG)]
    v = jnp.concatenate(rows, axis=0)

    iota = lax.broadcasted_iota(jnp.int32, (1, CHUNK), 1)
    mrows = []
    for qi in range(QG):
        segs = [iota + cid_ref[g * QG * K + qi * K + j] * CHUNK
                for j in range(K)]
        mrows.append(jnp.concatenate(segs, axis=1))
    memidx = jnp.concatenate(mrows, axis=0)

    v = jnp.where(memidx < MEM, v, NEG)
    for j in range(K):
        m = jnp.max(v, axis=1, keepdims=True)
        sel = jnp.min(jnp.where(v == m, memidx, IBIG), axis=1, keepdims=True)
        out_ref[:, j:j + 1] = sel
        v = jnp.where(memidx == sel, NEG, v)


def _gather_values(values, idx_flat):
    nw = 32
    bpw = idx_flat.shape[0] // nw
    mesh = plsc.VectorSubcoreMesh(core_axis_name="c", subcore_axis_name="s")

    @functools.partial(
        pl.kernel, mesh=mesh,
        compiler_params=pltpu.CompilerParams(use_tc_tiling_on_sc=False),
        out_type=jax.ShapeDtypeStruct((idx_flat.shape[0], H), jnp.float32),
        scratch_types=[
            pltpu.VMEM((bpw,), jnp.int32),
            pltpu.VMEM((bpw, H), jnp.float32),
            pltpu.SemaphoreType.DMA,
        ],
    )
    def gk(values_hbm, idx_hbm, out_hbm, idx_v, rows_v, sem):
        wid = lax.axis_index("s") * 2 + lax.axis_index("c")
        base = wid * bpw
        pltpu.sync_copy(idx_hbm.at[pl.ds(base, bpw)], idx_v)
        pltpu.async_copy(values_hbm.at[idx_v], rows_v, sem).wait()
        pltpu.sync_copy(rows_v, out_hbm.at[pl.ds(base, bpw)])

    return gk(values, idx_flat)


def _retrieve_idx(queries, keys):
    qn = queries / jnp.maximum(
        jnp.linalg.norm(queries, axis=1, keepdims=True), EPS)
    nmax = jnp.maximum(jnp.linalg.norm(keys, axis=1, keepdims=True), EPS)

    cmaxT, kn_pad = pl.pallas_call(
        _cmax_body,
        grid=(NB,),
        in_specs=[
            pl.BlockSpec((B, H), lambda i: (0, 0)),
            pl.BlockSpec((LBLK, H), lambda i: (i, 0)),
            pl.BlockSpec((LBLK, 1), lambda i: (i, 0)),
        ],
        out_specs=[
            pl.BlockSpec((CPB, B), lambda i: (i, 0)),
            pl.BlockSpec((LBLK, H), lambda i: (i, 0)),
        ],
        out_shape=[
            jax.ShapeDtypeStruct((NCHUNK, B), jnp.float32),
            jax.ShapeDtypeStruct((NB * LBLK, H), jnp.float32),
        ],
    )(qn, keys, nmax)

    cidsT = pl.pallas_call(
        _topchunk_body,
        grid=(8,),
        in_specs=[pl.BlockSpec((NCHUNK, 128), lambda g: (0, g))],
        out_specs=pl.BlockSpec((K, 128), lambda g: (0, g)),
        out_shape=jax.ShapeDtypeStruct((K, B), jnp.int32),
    )(cmaxT)

    cids_flat = cidsT.T.reshape(-1)

    key_specs = []
    for qi in range(QG):
        for j in range(K):
            def kmap(g, cid_ref, _qi=qi, _j=j):
                return (cid_ref[g * QG * K + _qi * K + _j], 0)
            key_specs.append(pl.BlockSpec((CHUNK, H), kmap))

    idx = pl.pallas_call(
        _cand_body,
        grid_spec=pltpu.PrefetchScalarGridSpec(
            num_scalar_prefetch=1,
            grid=(NGC,),
            in_specs=[pl.BlockSpec((QG, H), lambda g, c: (g, 0))]
            + key_specs,
            out_specs=pl.BlockSpec((QG, K), lambda g, c: (g, 0)),
        ),
        out_shape=jax.ShapeDtypeStruct((B, K), jnp.int32),
    )(cids_flat, qn, *([kn_pad] * (QG * K)))

    return idx


def kernel(queries, keys, values, top_num):
    idx = _retrieve_idx(queries, keys)
    rows = _gather_values(values, idx.reshape(-1))
    return rows.reshape(B, K, H)


# speedup vs baseline: 10.2460x; 1.4826x over previous
import functools

import jax
import jax.numpy as jnp
from jax import lax
from jax.experimental import pallas as pl
from jax.experimental.pallas import tpu as pltpu
from jax.experimental.pallas import tpu_sc as plsc

MEM = 1_000_000
H = 64
B = 1024
K = 8
LBLK = 1024
NB = 977
VALID_LAST = MEM - (NB - 1) * LBLK
CHUNK = 128
CPB = LBLK // CHUNK
NCHUNK = NB * CPB
QG = 8
NGC = B // QG
EPS = 1e-6
IBIG = 2**31 - 1

NEG = float("-inf")


def _cmax_body(qn_ref, keysT_ref, nmaxT_ref, out_ref, knT_ref):
    i = pl.program_id(0)
    knT = keysT_ref[...] / nmaxT_ref[...]
    knT_ref[...] = knT
    simsT = lax.dot_general(knT, qn_ref[...], (((0,), (1,)), ((), ())),
                            preferred_element_type=jnp.float32)

    def cmax(s):
        parts = [jnp.max(s[j * CHUNK:(j + 1) * CHUNK, :], axis=0, keepdims=True)
                 for j in range(CPB)]
        return jnp.concatenate(parts, axis=0)

    out_ref[...] = cmax(simsT)

    @pl.when(i == NB - 1)
    def _():
        row = lax.broadcasted_iota(jnp.int32, (LBLK, B), 0)
        out_ref[...] = cmax(jnp.where(row < VALID_LAST, simsT, NEG))


def _topchunk_body(cmax_ref, out_ref):
    v = cmax_ref[...]
    cidx = lax.broadcasted_iota(jnp.int32, (NCHUNK, 128), 0)
    for j in range(K):
        m = jnp.max(v, axis=0, keepdims=True)
        sel = jnp.min(jnp.where(v == m, cidx, IBIG), axis=0, keepdims=True)
        out_ref[j:j + 1, :] = sel
        v = jnp.where(cidx == sel, NEG, v)


def _cand_body(cid_ref, qn_ref, *refs):
    kn_refs = refs[:QG * K]
    out_ref = refs[QG * K]
    g = pl.program_id(0)

    kcat = jnp.concatenate([r[...] for r in kn_refs], axis=1)
    full = lax.dot_general(qn_ref[...], kcat, (((1,), (0,)), ((), ())),
                           preferred_element_type=jnp.float32)
    span = K * CHUNK
    rows = [full[qi:qi + 1, qi * span:(qi + 1) * span] for qi in range(QG)]
    v = jnp.concatenate(rows, axis=0)

    iota = lax.broadcasted_iota(jnp.int32, (1, CHUNK), 1)
    mrows = []
    for qi in range(QG):
        segs = [iota + cid_ref[g * QG * K + qi * K + j] * CHUNK
                for j in range(K)]
        mrows.append(jnp.concatenate(segs, axis=1))
    memidx = jnp.concatenate(mrows, axis=0)

    v = jnp.where(memidx < MEM, v, NEG)
    for j in range(K):
        m = jnp.max(v, axis=1, keepdims=True)
        sel = jnp.min(jnp.where(v == m, memidx, IBIG), axis=1, keepdims=True)
        out_ref[:, j:j + 1] = sel
        v = jnp.where(memidx == sel, NEG, v)


def _gather_values(values, idx_flat):
    nw = 32
    bpw = idx_flat.shape[0] // nw
    mesh = plsc.VectorSubcoreMesh(core_axis_name="c", subcore_axis_name="s")

    @functools.partial(
        pl.kernel, mesh=mesh,
        compiler_params=pltpu.CompilerParams(use_tc_tiling_on_sc=False),
        out_type=jax.ShapeDtypeStruct((idx_flat.shape[0], H), jnp.float32),
        scratch_types=[
            pltpu.VMEM((bpw,), jnp.int32),
            pltpu.VMEM((bpw, H), jnp.float32),
            pltpu.SemaphoreType.DMA,
        ],
    )
    def gk(values_hbm, idx_hbm, out_hbm, idx_v, rows_v, sem):
        wid = lax.axis_index("s") * 2 + lax.axis_index("c")
        base = wid * bpw
        pltpu.sync_copy(idx_hbm.at[pl.ds(base, bpw)], idx_v)
        pltpu.async_copy(values_hbm.at[idx_v], rows_v, sem).wait()
        pltpu.sync_copy(rows_v, out_hbm.at[pl.ds(base, bpw)])

    return gk(values, idx_flat)


def _retrieve_idx(queries, keys):
    qn = queries / jnp.maximum(
        jnp.linalg.norm(queries, axis=1, keepdims=True), EPS)
    nmax = jnp.maximum(jnp.linalg.norm(keys, axis=1, keepdims=True), EPS)
    keysT = keys.T
    nmaxT = nmax.reshape(1, MEM)

    cmaxT, knT_pad = pl.pallas_call(
        _cmax_body,
        grid=(NB,),
        in_specs=[
            pl.BlockSpec((B, H), lambda i: (0, 0)),
            pl.BlockSpec((H, LBLK), lambda i: (0, i)),
            pl.BlockSpec((1, LBLK), lambda i: (0, i)),
        ],
        out_specs=[
            pl.BlockSpec((CPB, B), lambda i: (i, 0)),
            pl.BlockSpec((H, LBLK), lambda i: (0, i)),
        ],
        out_shape=[
            jax.ShapeDtypeStruct((NCHUNK, B), jnp.float32),
            jax.ShapeDtypeStruct((H, NB * LBLK), jnp.float32),
        ],
    )(qn, keysT, nmaxT)

    cidsT = pl.pallas_call(
        _topchunk_body,
        grid=(8,),
        in_specs=[pl.BlockSpec((NCHUNK, 128), lambda g: (0, g))],
        out_specs=pl.BlockSpec((K, 128), lambda g: (0, g)),
        out_shape=jax.ShapeDtypeStruct((K, B), jnp.int32),
    )(cmaxT)

    cids_flat = cidsT.T.reshape(-1)

    key_specs = []
    for qi in range(QG):
        for j in range(K):
            def kmap(g, cid_ref, _qi=qi, _j=j):
                return (0, cid_ref[g * QG * K + _qi * K + _j])
            key_specs.append(pl.BlockSpec((H, CHUNK), kmap))

    idx = pl.pallas_call(
        _cand_body,
        grid_spec=pltpu.PrefetchScalarGridSpec(
            num_scalar_prefetch=1,
            grid=(NGC,),
            in_specs=[pl.BlockSpec((QG, H), lambda g, c: (g, 0))]
            + key_specs,
            out_specs=pl.BlockSpec((QG, K), lambda g, c: (g, 0)),
        ),
        out_shape=jax.ShapeDtypeStruct((B, K), jnp.int32),
    )(cids_flat, qn, *([knT_pad] * (QG * K)))

    return idx


def kernel(queries, keys, values, top_num):
    idx = _retrieve_idx(queries, keys)
    rows = _gather_values(values, idx.reshape(-1))
    return rows.reshape(B, K, H)


# speedup vs baseline: 12.4975x; 1.2197x over previous
import functools

import jax
import jax.numpy as jnp
from jax import lax
from jax.experimental import pallas as pl
from jax.experimental.pallas import tpu as pltpu
from jax.experimental.pallas import tpu_sc as plsc

MEM = 1_000_000
H = 64
B = 1024
K = 8
LBLK = 2048
NB = 489
VALID_LAST = MEM - (NB - 1) * LBLK
CHUNK = 128
CPB = LBLK // CHUNK
NCHUNK = NB * CPB
QG = 16
NGC = B // QG
EPS = 1e-6
IBIG = 2**31 - 1

NEG = float("-inf")


def _cmax_body(qn_ref, keysT_ref, nmaxT_ref, out_ref, knT_ref):
    i = pl.program_id(0)
    knT = keysT_ref[...] / nmaxT_ref[...]
    knT_ref[...] = knT
    simsT = lax.dot_general(knT, qn_ref[...], (((0,), (1,)), ((), ())),
                            preferred_element_type=jnp.float32)

    def cmax(s):
        parts = [jnp.max(s[j * CHUNK:(j + 1) * CHUNK, :], axis=0, keepdims=True)
                 for j in range(CPB)]
        return jnp.concatenate(parts, axis=0)

    out_ref[...] = cmax(simsT)

    @pl.when(i == NB - 1)
    def _():
        row = lax.broadcasted_iota(jnp.int32, (LBLK, B), 0)
        out_ref[...] = cmax(jnp.where(row < VALID_LAST, simsT, NEG))


def _topchunk_body(cmax_ref, out_ref):
    v = cmax_ref[...]
    cidx = lax.broadcasted_iota(jnp.int32, (NCHUNK, 128), 0)
    for j in range(K):
        m = jnp.max(v, axis=0, keepdims=True)
        sel = jnp.min(jnp.where(v == m, cidx, IBIG), axis=0, keepdims=True)
        out_ref[j:j + 1, :] = sel
        v = jnp.where(cidx == sel, NEG, v)


def _cand_body(cid_ref, qn_ref, *refs):
    kn_refs = refs[:QG * K]
    out_ref = refs[QG * K]
    g = pl.program_id(0)

    kcat = jnp.concatenate([r[...] for r in kn_refs], axis=1)
    full = lax.dot_general(qn_ref[...], kcat, (((1,), (0,)), ((), ())),
                           preferred_element_type=jnp.float32)
    span = K * CHUNK
    rows = [full[qi:qi + 1, qi * span:(qi + 1) * span] for qi in range(QG)]
    v = jnp.concatenate(rows, axis=0)

    iota = lax.broadcasted_iota(jnp.int32, (1, CHUNK), 1)
    mrows = []
    for qi in range(QG):
        segs = [iota + cid_ref[g * QG * K + qi * K + j] * CHUNK
                for j in range(K)]
        mrows.append(jnp.concatenate(segs, axis=1))
    memidx = jnp.concatenate(mrows, axis=0)

    v = jnp.where(memidx < MEM, v, NEG)
    for j in range(K):
        m = jnp.max(v, axis=1, keepdims=True)
        sel = jnp.min(jnp.where(v == m, memidx, IBIG), axis=1, keepdims=True)
        out_ref[:, j:j + 1] = sel
        v = jnp.where(memidx == sel, NEG, v)


def _gather_values(values, idx_flat):
    nw = 32
    bpw = idx_flat.shape[0] // nw
    mesh = plsc.VectorSubcoreMesh(core_axis_name="c", subcore_axis_name="s")

    @functools.partial(
        pl.kernel, mesh=mesh,
        compiler_params=pltpu.CompilerParams(use_tc_tiling_on_sc=False),
        out_type=jax.ShapeDtypeStruct((idx_flat.shape[0], H), jnp.float32),
        scratch_types=[
            pltpu.VMEM((bpw,), jnp.int32),
            pltpu.VMEM((bpw, H), jnp.float32),
            pltpu.SemaphoreType.DMA,
        ],
    )
    def gk(values_hbm, idx_hbm, out_hbm, idx_v, rows_v, sem):
        wid = lax.axis_index("s") * 2 + lax.axis_index("c")
        base = wid * bpw
        pltpu.sync_copy(idx_hbm.at[pl.ds(base, bpw)], idx_v)
        pltpu.async_copy(values_hbm.at[idx_v], rows_v, sem).wait()
        pltpu.sync_copy(rows_v, out_hbm.at[pl.ds(base, bpw)])

    return gk(values, idx_flat)


def _retrieve_idx(queries, keys):
    qn = queries / jnp.maximum(
        jnp.linalg.norm(queries, axis=1, keepdims=True), EPS)
    nmax = jnp.maximum(jnp.linalg.norm(keys, axis=1, keepdims=True), EPS)
    keysT = keys.T
    nmaxT = nmax.reshape(1, MEM)

    cmaxT, knT_pad = pl.pallas_call(
        _cmax_body,
        grid=(NB,),
        in_specs=[
            pl.BlockSpec((B, H), lambda i: (0, 0)),
            pl.BlockSpec((H, LBLK), lambda i: (0, i)),
            pl.BlockSpec((1, LBLK), lambda i: (0, i)),
        ],
        out_specs=[
            pl.BlockSpec((CPB, B), lambda i: (i, 0)),
            pl.BlockSpec((H, LBLK), lambda i: (0, i)),
        ],
        out_shape=[
            jax.ShapeDtypeStruct((NCHUNK, B), jnp.float32),
            jax.ShapeDtypeStruct((H, NB * LBLK), jnp.float32),
        ],
    )(qn, keysT, nmaxT)

    cidsT = pl.pallas_call(
        _topchunk_body,
        grid=(8,),
        in_specs=[pl.BlockSpec((NCHUNK, 128), lambda g: (0, g))],
        out_specs=pl.BlockSpec((K, 128), lambda g: (0, g)),
        out_shape=jax.ShapeDtypeStruct((K, B), jnp.int32),
    )(cmaxT)

    cids_flat = cidsT.T.reshape(-1)

    key_specs = []
    for qi in range(QG):
        for j in range(K):
            def kmap(g, cid_ref, _qi=qi, _j=j):
                return (0, cid_ref[g * QG * K + _qi * K + _j])
            key_specs.append(pl.BlockSpec((H, CHUNK), kmap))

    idx = pl.pallas_call(
        _cand_body,
        grid_spec=pltpu.PrefetchScalarGridSpec(
            num_scalar_prefetch=1,
            grid=(NGC,),
            in_specs=[pl.BlockSpec((QG, H), lambda g, c: (g, 0))]
            + key_specs,
            out_specs=pl.BlockSpec((QG, K), lambda g, c: (g, 0)),
        ),
        out_shape=jax.ShapeDtypeStruct((B, K), jnp.int32),
    )(cids_flat, qn, *([knT_pad] * (QG * K)))

    return idx


def kernel(queries, keys, values, top_num):
    idx = _retrieve_idx(queries, keys)
    rows = _gather_values(values, idx.reshape(-1))
    return rows.reshape(B, K, H)


# speedup vs baseline: 13.7773x; 1.1024x over previous
import functools

import jax
import jax.numpy as jnp
from jax import lax
from jax.experimental import pallas as pl
from jax.experimental.pallas import tpu as pltpu
from jax.experimental.pallas import tpu_sc as plsc

MEM = 1_000_000
H = 64
B = 1024
K = 8
LBLK = 2048
NB = 489
VALID_LAST = MEM - (NB - 1) * LBLK
CHUNK = 128
CPB = LBLK // CHUNK
NCHUNK = NB * CPB
QG = 16
NGC = B // QG
EPS = 1e-6
IBIG = 2**31 - 1

NEG = float("-inf")


def _cmax_body(qn_ref, keysT_ref, out_ref, knT_ref):
    i = pl.program_id(0)
    kt = keysT_ref[...]
    nrm = jnp.sqrt(jnp.sum(kt * kt, axis=0, keepdims=True))
    knT = kt / jnp.maximum(nrm, EPS)
    knT_ref[...] = knT
    simsT = lax.dot_general(knT, qn_ref[...], (((0,), (1,)), ((), ())),
                            preferred_element_type=jnp.float32)

    def cmax(s):
        parts = [jnp.max(s[j * CHUNK:(j + 1) * CHUNK, :], axis=0, keepdims=True)
                 for j in range(CPB)]
        return jnp.concatenate(parts, axis=0)

    out_ref[...] = cmax(simsT)

    @pl.when(i == NB - 1)
    def _():
        row = lax.broadcasted_iota(jnp.int32, (LBLK, B), 0)
        out_ref[...] = cmax(jnp.where(row < VALID_LAST, simsT, NEG))


def _topchunk_body(cmax_ref, out_ref):
    v = cmax_ref[...]
    cidx = lax.broadcasted_iota(jnp.int32, (NCHUNK, 128), 0)
    for j in range(K):
        m = jnp.max(v, axis=0, keepdims=True)
        sel = jnp.min(jnp.where(v == m, cidx, IBIG), axis=0, keepdims=True)
        out_ref[j:j + 1, :] = sel
        v = jnp.where(cidx == sel, NEG, v)


def _cand_body(cid_ref, qn_ref, *refs):
    kn_refs = refs[:QG * K]
    out_ref = refs[QG * K]
    g = pl.program_id(0)

    kcat = jnp.concatenate([r[...] for r in kn_refs], axis=1)
    full = lax.dot_general(qn_ref[...], kcat, (((1,), (0,)), ((), ())),
                           preferred_element_type=jnp.float32)
    span = K * CHUNK
    rows = [full[qi:qi + 1, qi * span:(qi + 1) * span] for qi in range(QG)]
    v = jnp.concatenate(rows, axis=0)

    iota = lax.broadcasted_iota(jnp.int32, (1, CHUNK), 1)
    mrows = []
    for qi in range(QG):
        segs = [iota + cid_ref[g * QG * K + qi * K + j] * CHUNK
                for j in range(K)]
        mrows.append(jnp.concatenate(segs, axis=1))
    memidx = jnp.concatenate(mrows, axis=0)

    v = jnp.where(memidx < MEM, v, NEG)
    for j in range(K):
        m = jnp.max(v, axis=1, keepdims=True)
        sel = jnp.min(jnp.where(v == m, memidx, IBIG), axis=1, keepdims=True)
        out_ref[:, j:j + 1] = sel
        v = jnp.where(memidx == sel, NEG, v)


def _gather_values(values, idx_flat):
    nw = 32
    bpw = idx_flat.shape[0] // nw
    mesh = plsc.VectorSubcoreMesh(core_axis_name="c", subcore_axis_name="s")

    @functools.partial(
        pl.kernel, mesh=mesh,
        compiler_params=pltpu.CompilerParams(use_tc_tiling_on_sc=False),
        out_type=jax.ShapeDtypeStruct((idx_flat.shape[0], H), jnp.float32),
        scratch_types=[
            pltpu.VMEM((bpw,), jnp.int32),
            pltpu.VMEM((bpw, H), jnp.float32),
            pltpu.SemaphoreType.DMA,
        ],
    )
    def gk(values_hbm, idx_hbm, out_hbm, idx_v, rows_v, sem):
        wid = lax.axis_index("s") * 2 + lax.axis_index("c")
        base = wid * bpw
        pltpu.sync_copy(idx_hbm.at[pl.ds(base, bpw)], idx_v)
        pltpu.async_copy(values_hbm.at[idx_v], rows_v, sem).wait()
        pltpu.sync_copy(rows_v, out_hbm.at[pl.ds(base, bpw)])

    return gk(values, idx_flat)


def _retrieve_idx(queries, keys):
    qn = queries / jnp.maximum(
        jnp.linalg.norm(queries, axis=1, keepdims=True), EPS)
    keysT = keys.T

    cmaxT, knT_pad = pl.pallas_call(
        _cmax_body,
        grid=(NB,),
        in_specs=[
            pl.BlockSpec((B, H), lambda i: (0, 0)),
            pl.BlockSpec((H, LBLK), lambda i: (0, i)),
        ],
        out_specs=[
            pl.BlockSpec((CPB, B), lambda i: (i, 0)),
            pl.BlockSpec((H, LBLK), lambda i: (0, i)),
        ],
        out_shape=[
            jax.ShapeDtypeStruct((NCHUNK, B), jnp.float32),
            jax.ShapeDtypeStruct((H, NB * LBLK), jnp.float32),
        ],
    )(qn, keysT)

    cidsT = pl.pallas_call(
        _topchunk_body,
        grid=(8,),
        in_specs=[pl.BlockSpec((NCHUNK, 128), lambda g: (0, g))],
        out_specs=pl.BlockSpec((K, 128), lambda g: (0, g)),
        out_shape=jax.ShapeDtypeStruct((K, B), jnp.int32),
    )(cmaxT)

    cids_flat = cidsT.T.reshape(-1)

    key_specs = []
    for qi in range(QG):
        for j in range(K):
            def kmap(g, cid_ref, _qi=qi, _j=j):
                return (0, cid_ref[g * QG * K + _qi * K + _j])
            key_specs.append(pl.BlockSpec((H, CHUNK), kmap))

    idx = pl.pallas_call(
        _cand_body,
        grid_spec=pltpu.PrefetchScalarGridSpec(
            num_scalar_prefetch=1,
            grid=(NGC,),
            in_specs=[pl.BlockSpec((QG, H), lambda g, c: (g, 0))]
            + key_specs,
            out_specs=pl.BlockSpec((QG, K), lambda g, c: (g, 0)),
        ),
        out_shape=jax.ShapeDtypeStruct((B, K), jnp.int32),
    )(cids_flat, qn, *([knT_pad] * (QG * K)))

    return idx


def kernel(queries, keys, values, top_num):
    idx = _retrieve_idx(queries, keys)
    rows = _gather_values(values, idx.reshape(-1))
    return rows.reshape(B, K, H)


# speedup vs baseline: 20.9546x; 1.5210x over previous
import functools

import jax
import jax.numpy as jnp
from jax import lax
from jax.experimental import pallas as pl
from jax.experimental.pallas import tpu as pltpu
from jax.experimental.pallas import tpu_sc as plsc

MEM = 1_000_000
H = 64
B = 1024
K = 8
LBLK = 4096
NB = 245
VALID_LAST = MEM - (NB - 1) * LBLK
CHUNK = 128
CPB = LBLK // CHUNK
NCHUNK = NB * CPB
QG = 32
NGC = B // QG
EPS = 1e-6
IBIG = 2**31 - 1

NEG = float("-inf")


def _cmax_body(qn_ref, keysT_ref, out_ref, knT_ref):
    i = pl.program_id(0)
    kt = keysT_ref[...]
    nrm = jnp.sqrt(jnp.sum(kt * kt, axis=0, keepdims=True))
    knT = kt / jnp.maximum(nrm, EPS)
    knT_ref[...] = knT
    simsT = lax.dot_general(knT, qn_ref[...], (((0,), (1,)), ((), ())),
                            preferred_element_type=jnp.float32)

    def cmax(s):
        parts = [jnp.max(s[j * CHUNK:(j + 1) * CHUNK, :], axis=0, keepdims=True)
                 for j in range(CPB)]
        return jnp.concatenate(parts, axis=0)

    out_ref[...] = cmax(simsT)

    @pl.when(i == NB - 1)
    def _():
        row = lax.broadcasted_iota(jnp.int32, (LBLK, B), 0)
        out_ref[...] = cmax(jnp.where(row < VALID_LAST, simsT, NEG))


def _topchunk_body(cmax_ref, out_ref):
    v = cmax_ref[...]
    cidx = lax.broadcasted_iota(jnp.int32, (NCHUNK, 128), 0)
    for j in range(K):
        m = jnp.max(v, axis=0, keepdims=True)
        sel = jnp.min(jnp.where(v == m, cidx, IBIG), axis=0, keepdims=True)
        out_ref[j:j + 1, :] = sel
        v = jnp.where(cidx == sel, NEG, v)


def _cand_body(cid_ref, qn_ref, *refs):
    kn_refs = refs[:QG * K]
    out_ref = refs[QG * K]
    g = pl.program_id(0)

    kcat = jnp.concatenate([r[...] for r in kn_refs], axis=1)
    full = lax.dot_general(qn_ref[...], kcat, (((1,), (0,)), ((), ())),
                           preferred_element_type=jnp.float32)
    span = K * CHUNK
    rows = [full[qi:qi + 1, qi * span:(qi + 1) * span] for qi in range(QG)]
    v = jnp.concatenate(rows, axis=0)

    iota = lax.broadcasted_iota(jnp.int32, (1, CHUNK), 1)
    mrows = []
    for qi in range(QG):
        segs = [iota + cid_ref[g * QG * K + qi * K + j] * CHUNK
                for j in range(K)]
        mrows.append(jnp.concatenate(segs, axis=1))
    memidx = jnp.concatenate(mrows, axis=0)

    v = jnp.where(memidx < MEM, v, NEG)
    for j in range(K):
        m = jnp.max(v, axis=1, keepdims=True)
        sel = jnp.min(jnp.where(v == m, memidx, IBIG), axis=1, keepdims=True)
        out_ref[:, j:j + 1] = sel
        v = jnp.where(memidx == sel, NEG, v)


def _gather_values(values, idx_flat):
    nw = 32
    bpw = idx_flat.shape[0] // nw
    mesh = plsc.VectorSubcoreMesh(core_axis_name="c", subcore_axis_name="s")

    @functools.partial(
        pl.kernel, mesh=mesh,
        compiler_params=pltpu.CompilerParams(use_tc_tiling_on_sc=False),
        out_type=jax.ShapeDtypeStruct((idx_flat.shape[0], H), jnp.float32),
        scratch_types=[
            pltpu.VMEM((bpw,), jnp.int32),
            pltpu.VMEM((bpw, H), jnp.float32),
            pltpu.SemaphoreType.DMA,
        ],
    )
    def gk(values_hbm, idx_hbm, out_hbm, idx_v, rows_v, sem):
        wid = lax.axis_index("s") * 2 + lax.axis_index("c")
        base = wid * bpw
        pltpu.sync_copy(idx_hbm.at[pl.ds(base, bpw)], idx_v)
        pltpu.async_copy(values_hbm.at[idx_v], rows_v, sem).wait()
        pltpu.sync_copy(rows_v, out_hbm.at[pl.ds(base, bpw)])

    return gk(values, idx_flat)


def _retrieve_idx(queries, keys):
    qn = queries / jnp.maximum(
        jnp.linalg.norm(queries, axis=1, keepdims=True), EPS)
    keysT = keys.T

    cmaxT, knT_pad = pl.pallas_call(
        _cmax_body,
        grid=(NB,),
        in_specs=[
            pl.BlockSpec((B, H), lambda i: (0, 0)),
            pl.BlockSpec((H, LBLK), lambda i: (0, i)),
        ],
        out_specs=[
            pl.BlockSpec((CPB, B), lambda i: (i, 0)),
            pl.BlockSpec((H, LBLK), lambda i: (0, i)),
        ],
        out_shape=[
            jax.ShapeDtypeStruct((NCHUNK, B), jnp.float32),
            jax.ShapeDtypeStruct((H, NB * LBLK), jnp.float32),
        ],
    )(qn, keysT)

    cidsT = pl.pallas_call(
        _topchunk_body,
        grid=(8,),
        in_specs=[pl.BlockSpec((NCHUNK, 128), lambda g: (0, g))],
        out_specs=pl.BlockSpec((K, 128), lambda g: (0, g)),
        out_shape=jax.ShapeDtypeStruct((K, B), jnp.int32),
    )(cmaxT)

    cids_flat = cidsT.T.reshape(-1)

    key_specs = []
    for qi in range(QG):
        for j in range(K):
            def kmap(g, cid_ref, _qi=qi, _j=j):
                return (0, cid_ref[g * QG * K + _qi * K + _j])
            key_specs.append(pl.BlockSpec((H, CHUNK), kmap))

    idx = pl.pallas_call(
        _cand_body,
        grid_spec=pltpu.PrefetchScalarGridSpec(
            num_scalar_prefetch=1,
            grid=(NGC,),
            in_specs=[pl.BlockSpec((QG, H), lambda g, c: (g, 0))]
            + key_specs,
            out_specs=pl.BlockSpec((QG, K), lambda g, c: (g, 0)),
        ),
        out_shape=jax.ShapeDtypeStruct((B, K), jnp.int32),
    )(cids_flat, qn, *([knT_pad] * (QG * K)))

    return idx


def kernel(queries, keys, values, top_num):
    idx = _retrieve_idx(queries, keys)
    rows = _gather_values(values, idx.reshape(-1))
    return rows.reshape(B, K, H)


# speedup vs baseline: 20.9569x; 1.0001x over previous
import functools

import jax
import jax.numpy as jnp
from jax import lax
from jax.experimental import pallas as pl
from jax.experimental.pallas import tpu as pltpu
from jax.experimental.pallas import tpu_sc as plsc

MEM = 1_000_000
H = 64
B = 1024
K = 8
LBLK = 4096
NB = 245
VALID_LAST = MEM - (NB - 1) * LBLK
CHUNK = 128
CPB = LBLK // CHUNK
NCHUNK = NB * CPB
QG = 32
NGC = B // QG
EPS = 1e-6
IBIG = 2**31 - 1

NEG = float("-inf")


def _cmax_body(qn_ref, keysT_ref, valsT_ref, out_ref, knT_ref, vp_ref):
    i = pl.program_id(0)
    kt = keysT_ref[...]
    nrm = jnp.sqrt(jnp.sum(kt * kt, axis=0, keepdims=True))
    knT = kt / jnp.maximum(nrm, EPS)
    knT_ref[...] = knT
    vals = valsT_ref[...].T
    vp_ref[:, 0:H] = vals[0:LBLK // 2, :]
    vp_ref[:, H:2 * H] = vals[LBLK // 2:LBLK, :]
    simsT = lax.dot_general(knT, qn_ref[...], (((0,), (1,)), ((), ())),
                            preferred_element_type=jnp.float32)

    def cmax(s):
        parts = [jnp.max(s[j * CHUNK:(j + 1) * CHUNK, :], axis=0, keepdims=True)
                 for j in range(CPB)]
        return jnp.concatenate(parts, axis=0)

    out_ref[...] = cmax(simsT)

    @pl.when(i == NB - 1)
    def _():
        row = lax.broadcasted_iota(jnp.int32, (LBLK, B), 0)
        out_ref[...] = cmax(jnp.where(row < VALID_LAST, simsT, NEG))


def _topchunk_body(cmax_ref, out_ref):
    v = cmax_ref[...]
    cidx = lax.broadcasted_iota(jnp.int32, (NCHUNK, 128), 0)
    for j in range(K):
        m = jnp.max(v, axis=0, keepdims=True)
        sel = jnp.min(jnp.where(v == m, cidx, IBIG), axis=0, keepdims=True)
        out_ref[j:j + 1, :] = sel
        v = jnp.where(cidx == sel, NEG, v)


def _cand_body(cid_ref, qn_ref, *refs):
    kn_refs = refs[:QG * K]
    out_ref = refs[QG * K]
    g = pl.program_id(0)

    kcat = jnp.concatenate([r[...] for r in kn_refs], axis=1)
    full = lax.dot_general(qn_ref[...], kcat, (((1,), (0,)), ((), ())),
                           preferred_element_type=jnp.float32)
    span = K * CHUNK
    rows = [full[qi:qi + 1, qi * span:(qi + 1) * span] for qi in range(QG)]
    v = jnp.concatenate(rows, axis=0)

    iota = lax.broadcasted_iota(jnp.int32, (1, CHUNK), 1)
    mrows = []
    for qi in range(QG):
        segs = [iota + cid_ref[g * QG * K + qi * K + j] * CHUNK
                for j in range(K)]
        mrows.append(jnp.concatenate(segs, axis=1))
    memidx = jnp.concatenate(mrows, axis=0)

    v = jnp.where(memidx < MEM, v, NEG)
    for j in range(K):
        m = jnp.max(v, axis=1, keepdims=True)
        sel = jnp.min(jnp.where(v == m, memidx, IBIG), axis=1, keepdims=True)
        out_ref[:, j:j + 1] = sel
        v = jnp.where(memidx == sel, NEG, v)


def _gather_values(values_pairs, pid_flat):
    n = pid_flat.shape[0]
    nw = 32
    bpw = n // nw
    mesh = plsc.VectorSubcoreMesh(core_axis_name="c", subcore_axis_name="s")

    @functools.partial(
        pl.kernel, mesh=mesh,
        out_type=jax.ShapeDtypeStruct((n, 2 * H), jnp.float32),
        scratch_types=[
            pltpu.VMEM((bpw,), jnp.int32),
            pltpu.VMEM((bpw, 2 * H), jnp.float32),
            pltpu.SemaphoreType.DMA,
        ],
    )
    def gk(values_hbm, pid_hbm, out_hbm, pid_v, rows_v, sem):
        wid = lax.axis_index("s") * 2 + lax.axis_index("c")
        base = wid * bpw
        pltpu.sync_copy(pid_hbm.at[pl.ds(base, bpw)], pid_v)
        pltpu.async_copy(values_hbm.at[pid_v], rows_v, sem).wait()
        pltpu.sync_copy(rows_v, out_hbm.at[pl.ds(base, bpw)])

    return gk(values_pairs, pid_flat)


def _retrieve_idx(queries, keys, values):
    qn = queries / jnp.maximum(
        jnp.linalg.norm(queries, axis=1, keepdims=True), EPS)
    keysT = keys.T
    valsT = values.T

    cmaxT, knT_pad, vpairs = pl.pallas_call(
        _cmax_body,
        grid=(NB,),
        in_specs=[
            pl.BlockSpec((B, H), lambda i: (0, 0)),
            pl.BlockSpec((H, LBLK), lambda i: (0, i)),
            pl.BlockSpec((H, LBLK), lambda i: (0, i)),
        ],
        out_specs=[
            pl.BlockSpec((CPB, B), lambda i: (i, 0)),
            pl.BlockSpec((H, LBLK), lambda i: (0, i)),
            pl.BlockSpec((LBLK // 2, 2 * H), lambda i: (i, 0)),
        ],
        out_shape=[
            jax.ShapeDtypeStruct((NCHUNK, B), jnp.float32),
            jax.ShapeDtypeStruct((H, NB * LBLK), jnp.float32),
            jax.ShapeDtypeStruct((NB * LBLK // 2, 2 * H), jnp.float32),
        ],
    )(qn, keysT, valsT)

    cidsT = pl.pallas_call(
        _topchunk_body,
        grid=(8,),
        in_specs=[pl.BlockSpec((NCHUNK, 128), lambda g: (0, g))],
        out_specs=pl.BlockSpec((K, 128), lambda g: (0, g)),
        out_shape=jax.ShapeDtypeStruct((K, B), jnp.int32),
    )(cmaxT)

    cids_flat = cidsT.T.reshape(-1)

    key_specs = []
    for qi in range(QG):
        for j in range(K):
            def kmap(g, cid_ref, _qi=qi, _j=j):
                return (0, cid_ref[g * QG * K + _qi * K + _j])
            key_specs.append(pl.BlockSpec((H, CHUNK), kmap))

    idx = pl.pallas_call(
        _cand_body,
        grid_spec=pltpu.PrefetchScalarGridSpec(
            num_scalar_prefetch=1,
            grid=(NGC,),
            in_specs=[pl.BlockSpec((QG, H), lambda g, c: (g, 0))]
            + key_specs,
            out_specs=pl.BlockSpec((QG, K), lambda g, c: (g, 0)),
        ),
        out_shape=jax.ShapeDtypeStruct((B, K), jnp.int32),
    )(cids_flat, qn, *([knT_pad] * (QG * K)))

    return idx, vpairs


def kernel(queries, keys, values, top_num):
    idx, vpairs = _retrieve_idx(queries, keys, values)
    idx_flat = idx.reshape(-1)
    blk = idx_flat // LBLK
    r = idx_flat % LBLK
    pid = blk * (LBLK // 2) + (r % (LBLK // 2))
    pairs = _gather_values(vpairs, pid)
    back = (r // (LBLK // 2)).reshape(B * K, 1) == 1
    rows = jnp.where(back, pairs[:, H:], pairs[:, :H])
    return rows.reshape(B, K, H)


# speedup vs baseline: 21.8437x; 1.0423x over previous
import functools

import jax
import jax.numpy as jnp
from jax import lax
from jax.experimental import pallas as pl
from jax.experimental.pallas import tpu as pltpu
from jax.experimental.pallas import tpu_sc as plsc

MEM = 1_000_000
H = 64
B = 1024
K = 8
LBLK = 8192
NB = 123
VALID_LAST = MEM - (NB - 1) * LBLK
CHUNK = 128
CPB = LBLK // CHUNK
NCHUNK = NB * CPB
QG = 32
NGC = B // QG
EPS = 1e-6
IBIG = 2**31 - 1

NEG = float("-inf")


def _cmax_body(qn_ref, keysT_ref, valsT_ref, out_ref, knT_ref, vp_ref):
    i = pl.program_id(0)
    kt = keysT_ref[...]
    nrm = jnp.sqrt(jnp.sum(kt * kt, axis=0, keepdims=True))
    knT = kt / jnp.maximum(nrm, EPS)
    knT_ref[...] = knT
    vals = valsT_ref[...].T
    vp_ref[:, 0:H] = vals[0:LBLK // 2, :]
    vp_ref[:, H:2 * H] = vals[LBLK // 2:LBLK, :]
    simsT = lax.dot_general(knT, qn_ref[...], (((0,), (1,)), ((), ())),
                            preferred_element_type=jnp.float32)

    def cmax(s):
        parts = [jnp.max(s[j * CHUNK:(j + 1) * CHUNK, :], axis=0, keepdims=True)
                 for j in range(CPB)]
        return jnp.concatenate(parts, axis=0)

    out_ref[...] = cmax(simsT)

    @pl.when(i == NB - 1)
    def _():
        row = lax.broadcasted_iota(jnp.int32, (LBLK, B), 0)
        out_ref[...] = cmax(jnp.where(row < VALID_LAST, simsT, NEG))


def _topchunk_body(cmax_ref, out_ref):
    v = cmax_ref[...]
    cidx = lax.broadcasted_iota(jnp.int32, (NCHUNK, 128), 0)
    for j in range(K):
        m = jnp.max(v, axis=0, keepdims=True)
        sel = jnp.min(jnp.where(v == m, cidx, IBIG), axis=0, keepdims=True)
        out_ref[j:j + 1, :] = sel
        v = jnp.where(cidx == sel, NEG, v)


def _cand_body(cid_ref, qn_ref, *refs):
    kn_refs = refs[:QG * K]
    out_ref = refs[QG * K]
    g = pl.program_id(0)

    kcat = jnp.concatenate([r[...] for r in kn_refs], axis=1)
    full = lax.dot_general(qn_ref[...], kcat, (((1,), (0,)), ((), ())),
                           preferred_element_type=jnp.float32)
    span = K * CHUNK
    rows = [full[qi:qi + 1, qi * span:(qi + 1) * span] for qi in range(QG)]
    v = jnp.concatenate(rows, axis=0)

    iota = lax.broadcasted_iota(jnp.int32, (1, CHUNK), 1)
    mrows = []
    for qi in range(QG):
        segs = [iota + cid_ref[g * QG * K + qi * K + j] * CHUNK
                for j in range(K)]
        mrows.append(jnp.concatenate(segs, axis=1))
    memidx = jnp.concatenate(mrows, axis=0)

    v = jnp.where(memidx < MEM, v, NEG)
    for j in range(K):
        m = jnp.max(v, axis=1, keepdims=True)
        sel = jnp.min(jnp.where(v == m, memidx, IBIG), axis=1, keepdims=True)
        out_ref[:, j:j + 1] = sel
        v = jnp.where(memidx == sel, NEG, v)


def _gather_values(values_pairs, pid_flat):
    n = pid_flat.shape[0]
    nw = 32
    bpw = n // nw
    mesh = plsc.VectorSubcoreMesh(core_axis_name="c", subcore_axis_name="s")

    @functools.partial(
        pl.kernel, mesh=mesh,
        out_type=jax.ShapeDtypeStruct((n, 2 * H), jnp.float32),
        scratch_types=[
            pltpu.VMEM((bpw,), jnp.int32),
            pltpu.VMEM((bpw, 2 * H), jnp.float32),
            pltpu.SemaphoreType.DMA,
        ],
    )
    def gk(values_hbm, pid_hbm, out_hbm, pid_v, rows_v, sem):
        wid = lax.axis_index("s") * 2 + lax.axis_index("c")
        base = wid * bpw
        pltpu.sync_copy(pid_hbm.at[pl.ds(base, bpw)], pid_v)
        pltpu.async_copy(values_hbm.at[pid_v], rows_v, sem).wait()
        pltpu.sync_copy(rows_v, out_hbm.at[pl.ds(base, bpw)])

    return gk(values_pairs, pid_flat)


def _retrieve_idx(queries, keys, values):
    qn = queries / jnp.maximum(
        jnp.linalg.norm(queries, axis=1, keepdims=True), EPS)
    keysT = keys.T
    valsT = values.T

    cmaxT, knT_pad, vpairs = pl.pallas_call(
        _cmax_body,
        grid=(NB,),
        in_specs=[
            pl.BlockSpec((B, H), lambda i: (0, 0)),
            pl.BlockSpec((H, LBLK), lambda i: (0, i)),
            pl.BlockSpec((H, LBLK), lambda i: (0, i)),
        ],
        out_specs=[
            pl.BlockSpec((CPB, B), lambda i: (i, 0)),
            pl.BlockSpec((H, LBLK), lambda i: (0, i)),
            pl.BlockSpec((LBLK // 2, 2 * H), lambda i: (i, 0)),
        ],
        out_shape=[
            jax.ShapeDtypeStruct((NCHUNK, B), jnp.float32),
            jax.ShapeDtypeStruct((H, NB * LBLK), jnp.float32),
            jax.ShapeDtypeStruct((NB * LBLK // 2, 2 * H), jnp.float32),
        ],
    )(qn, keysT, valsT)

    cidsT = pl.pallas_call(
        _topchunk_body,
        grid=(8,),
        in_specs=[pl.BlockSpec((NCHUNK, 128), lambda g: (0, g))],
        out_specs=pl.BlockSpec((K, 128), lambda g: (0, g)),
        out_shape=jax.ShapeDtypeStruct((K, B), jnp.int32),
    )(cmaxT)

    cids_flat = cidsT.T.reshape(-1)

    key_specs = []
    for qi in range(QG):
        for j in range(K):
            def kmap(g, cid_ref, _qi=qi, _j=j):
                return (0, cid_ref[g * QG * K + _qi * K + _j])
            key_specs.append(pl.BlockSpec((H, CHUNK), kmap))

    idx = pl.pallas_call(
        _cand_body,
        grid_spec=pltpu.PrefetchScalarGridSpec(
            num_scalar_prefetch=1,
            grid=(NGC,),
            in_specs=[pl.BlockSpec((QG, H), lambda g, c: (g, 0))]
            + key_specs,
            out_specs=pl.BlockSpec((QG, K), lambda g, c: (g, 0)),
        ),
        out_shape=jax.ShapeDtypeStruct((B, K), jnp.int32),
    )(cids_flat, qn, *([knT_pad] * (QG * K)))

    return idx, vpairs


def kernel(queries, keys, values, top_num):
    idx, vpairs = _retrieve_idx(queries, keys, values)
    idx_flat = idx.reshape(-1)
    blk = idx_flat // LBLK
    r = idx_flat % LBLK
    pid = blk * (LBLK // 2) + (r % (LBLK // 2))
    pairs = _gather_values(vpairs, pid)
    back = (r // (LBLK // 2)).reshape(B * K, 1) == 1
    rows = jnp.where(back, pairs[:, H:], pairs[:, :H])
    return rows.reshape(B, K, H)
